# Initial kernel scaffold; baseline (speedup 1.0000x reference)
import jax
import jax.numpy as jnp
from jax.experimental import pallas as pl


def kernel(mm_f_edges, mm_f_mat, mm_s_edges, mm_s_mat, dd_f_edges, dd_f_mat, dd_s_edges, dd_s_mat, x_m, x_d, Wx1f, bx1f, Wx2f, bx2f, Wx1s, bx1s, Wx2s, bx2s, Wy1f, by1f, Wy2f, by2f, Wy1s, by1s, Wy2s, by2s, Wfc1x, bfc1x, Wfc2x, bfc2x, Wfc1y, bfc1y, Wfc2y, bfc2y, Wcx, bcx, Wcy, bcy):
    raise NotImplementedError("write your pallas kernel here")


# speedup vs baseline: 7.6684x; 7.6684x over previous
import functools

import jax
import jax.numpy as jnp
from jax import lax
from jax.experimental import pallas as pl
from jax.experimental.pallas import tpu as pltpu
from jax.experimental.pallas import tpu_sc as plsc

F = 128
N = 5000
NPAD = 5120
E = 160000
OC = 128
CH = 128
NW = 32
CPW = 40
NCH = E // CH
RPT = NPAD // 16

def _zero_vec(ref, nwords):
    def body(i, _):
        ref[pl.ds(i * 16, 16)] = jnp.zeros((16,), jnp.float32)
        return 0
    lax.fori_loop(0, nwords // 16, body, 0)


def _zero_rows(ref, nrows):
    def body(i, _):
        for j in range(F // 16):
            ref[i, pl.ds(j * 16, 16)] = jnp.zeros((16,), jnp.float32)
        return 0
    lax.fori_loop(0, nrows, body, 0)


@functools.cache
def _make_s1():
    mesh = plsc.VectorSubcoreMesh(core_axis_name="c", subcore_axis_name="s")
    return functools.partial(
        pl.kernel,
        mesh=mesh,
        out_type=[
            jax.ShapeDtypeStruct((4 * E,), jnp.float32),
            jax.ShapeDtypeStruct((8 * NPAD,), jnp.float32),
        ],
        scratch_types=[
            pltpu.VMEM((CH,), jnp.int32),
            pltpu.VMEM((CH,), jnp.int32),
            pltpu.VMEM((CH,), jnp.int32),
            pltpu.VMEM((CH,), jnp.float32),
            pltpu.VMEM((RPT,), jnp.float32),
            pltpu.VMEM_SHARED((NPAD,), jnp.float32),
            pltpu.VMEM_SHARED((NPAD,), jnp.float32),
            pltpu.VMEM_SHARED((NPAD,), jnp.float32),
            pltpu.VMEM_SHARED((NPAD,), jnp.float32),
            pltpu.SemaphoreType.DMA,
        ],
    )(_s1_body)


def _s1_body(e0, m0, e1, m1, e2, m2, e3, m3,
             ew_out, deg_out,
             srcv, dstv, idxv, ewv, zv, dg0, dg1, dg2, dg3, sem):
    deg_sh = (dg0, dg1, dg2, dg3)
    c = lax.axis_index("c")
    s = lax.axis_index("s")
    w = s * 2 + c

    _zero_vec(zv, RPT)
    for g in range(4):
        pltpu.sync_copy(zv, deg_sh[g].at[pl.ds(s * RPT, RPT)])
    plsc.subcore_barrier()

    for g, (ed, mat) in enumerate(((e0, m0), (e1, m1), (e2, m2), (e3, m3))):
        def chunk(i, _, ed=ed, mat=mat, g=g):
            cid = w * CPW + i

            @pl.when(cid < NCH)
            def _():
                base = cid * CH
                pltpu.sync_copy(ed.at[pl.ds(base, CH)], srcv)
                pltpu.sync_copy(ed.at[pl.ds(E + base, CH)], dstv)

                def jb(j, _):
                    sl = pl.ds(j * 16, 16)
                    idxv[sl] = srcv[sl] * N + dstv[sl]
                    return 0
                lax.fori_loop(0, CH // 16, jb, 0)
                pltpu.async_copy(mat.at[idxv], ewv, sem).wait()
                pltpu.sync_copy(ewv, ew_out.at[pl.ds(g * E + base, CH)])
                pltpu.sync_copy(ewv, deg_sh[g].at[dstv], add=True)
            return 0
        lax.fori_loop(0, CPW, chunk, 0)

    plsc.subcore_barrier()
    for g in range(4):
        pltpu.sync_copy(deg_sh[g].at[pl.ds(s * RPT, RPT)], zv)
        pltpu.sync_copy(zv, deg_out.at[pl.ds((c * 4 + g) * NPAD + s * RPT, RPT)])


@functools.cache
def _make_s2():
    mesh = plsc.VectorSubcoreMesh(core_axis_name="c", subcore_axis_name="s")
    return functools.partial(
        pl.kernel,
        mesh=mesh,
        out_type=jax.ShapeDtypeStruct((2, 4, NPAD, F), jnp.float32),
        scratch_types=[
            pltpu.VMEM((CH,), jnp.int32),
            pltpu.VMEM((CH,), jnp.int32),
            pltpu.VMEM((CH,), jnp.float32),
            pltpu.VMEM((CH, F), jnp.float32),
            pltpu.VMEM((RPT, F), jnp.float32),
            pltpu.VMEM_SHARED((NPAD, F), jnp.float32),
            pltpu.SemaphoreType.DMA,
        ],
    )(_s2_body)


def _s2_body(tabs, e0, e1, e2, e3, ews,
             out,
             srcv, dstv, ewv, rows, zrows, acc_sh, sem):
    c = lax.axis_index("c")
    s = lax.axis_index("s")
    w = s * 2 + c

    _zero_rows(zrows, RPT)

    for g, ed in enumerate((e0, e1, e2, e3)):
        pltpu.sync_copy(zrows, acc_sh.at[pl.ds(s * RPT, RPT)])
        plsc.subcore_barrier()

        def chunk(i, _, ed=ed, g=g):
            cid = w * CPW + i

            @pl.when(cid < NCH)
            def _():
                base = cid * CH
                pltpu.sync_copy(ed.at[pl.ds(base, CH)], srcv)
                pltpu.sync_copy(ed.at[pl.ds(E + base, CH)], dstv)
                pltpu.sync_copy(ews.at[pl.ds(g * E + base, CH)], ewv)
                pltpu.async_copy(tabs.at[g].at[srcv], rows, sem).wait()

                def eb(k, _):
                    ew16 = ewv[pl.ds(k * 16, 16)]
                    for l in range(16):
                        wv = jnp.full((16,), ew16[l], jnp.float32)
                        e = k * 16 + l
                        for j in range(F // 16):
                            sl = pl.ds(j * 16, 16)
                            rows[e, sl] = rows[e, sl] * wv
                    return 0
                lax.fori_loop(0, CH // 16, eb, 0)
                pltpu.sync_copy(rows, acc_sh.at[dstv], add=True)
            return 0
        lax.fori_loop(0, CPW, chunk, 0)

        plsc.subcore_barrier()
        pltpu.sync_copy(acc_sh.at[pl.ds(s * RPT, RPT)],
                        out.at[c, g, pl.ds(s * RPT, RPT)])
        plsc.subcore_barrier()


def _mm(a, b):
    M, K = a.shape
    Nn = b.shape[1]
    bm = 512
    bn = min(Nn, 512)

    def body(ar, br, orf):
        orf[...] = jnp.dot(ar[...], br[...],
                           preferred_element_type=jnp.float32)
    return pl.pallas_call(
        body,
        grid=(pl.cdiv(M, bm), pl.cdiv(Nn, bn)),
        in_specs=[
            pl.BlockSpec((bm, K), lambda i, j: (i, 0)),
            pl.BlockSpec((K, bn), lambda i, j: (0, j)),
        ],
        out_specs=pl.BlockSpec((bm, bn), lambda i, j: (i, j)),
        out_shape=jax.ShapeDtypeStruct((M, Nn), jnp.float32),
    )(a, b)


def _bmm(a, b):
    B, M, K = a.shape
    Nn = b.shape[2]
    bm = 512

    def body(ar, br, orf):
        orf[0] = jnp.dot(ar[0], br[0], preferred_element_type=jnp.float32)
    return pl.pallas_call(
        body,
        grid=(B, pl.cdiv(M, bm)),
        in_specs=[
            pl.BlockSpec((1, bm, K), lambda g, i: (g, i, 0)),
            pl.BlockSpec((1, K, Nn), lambda g, i: (g, 0, 0)),
        ],
        out_specs=pl.BlockSpec((1, bm, Nn), lambda g, i: (g, i, 0)),
        out_shape=jax.ShapeDtypeStruct((B, M, Nn), jnp.float32),
    )(a, b)


def _attn(feats4, Wfc1, bfc1, Wfc2, bfc2, Wc, bc):
    a = feats4.mean(axis=(1, 2))
    a = jax.nn.relu(a @ Wfc1.T + bfc1)
    a = jax.nn.sigmoid(a @ Wfc2.T + bfc2)
    A = (a[:, None, None] * Wc[..., 0].transpose(1, 2, 0)).reshape(4 * F, OC)
    Fcat = feats4.transpose(1, 0, 2).reshape(N, 4 * F)
    return _mm(Fcat, A) + bc[None, :]


def kernel(mm_f_edges, mm_f_mat, mm_s_edges, mm_s_mat, dd_f_edges, dd_f_mat,
           dd_s_edges, dd_s_mat, x_m, x_d,
           Wx1f, bx1f, Wx2f, bx2f, Wx1s, bx1s, Wx2s, bx2s,
           Wy1f, by1f, Wy2f, by2f, Wy1s, by1s, Wy2s, by2s,
           Wfc1x, bfc1x, Wfc2x, bfc2x, Wfc1y, bfc1y, Wfc2y, bfc2y,
           Wcx, bcx, Wcy, bcy):
    edges = tuple(e.reshape(-1) for e in
                  (mm_f_edges, mm_s_edges, dd_f_edges, dd_s_edges))
    mats = (mm_f_mat.reshape(-1), mm_s_mat.reshape(-1),
            dd_f_mat.reshape(-1), dd_s_mat.reshape(-1))

    ew, degp = _make_s1()(edges[0], mats[0], edges[1], mats[1],
                          edges[2], mats[2], edges[3], mats[3])
    degp = degp.reshape(2, 4, NPAD)
    deg = degp[0] + degp[1] + 1.0
    dinv = jax.lax.rsqrt(deg)[:, :N]
    dcol = dinv[:, :, None]

    X0 = jnp.stack([x_m, x_m, x_d, x_d])
    W1 = jnp.stack([Wx1f, Wx1s, Wy1f, Wy1s]).transpose(0, 2, 1)
    b1 = jnp.stack([bx1f, bx1s, by1f, by1s])
    W2 = jnp.stack([Wx2f, Wx2s, Wy2f, Wy2s]).transpose(0, 2, 1)
    b2 = jnp.stack([bx2f, bx2s, by2f, by2s])

    def layer(xin, Wt, b):
        h = _bmm(xin, Wt)
        t = dcol * h
        aggp = _make_s2()(t, edges[0], edges[1], edges[2], edges[3], ew)
        agg = aggp[0, :, :N] + aggp[1, :, :N]
        return jax.nn.relu(dcol * (agg + t) + b[:, None, :])

    h1 = layer(X0, W1, b1)
    h2 = layer(h1, W2, b2)

    featx = jnp.stack([h1[0], h2[0], h1[1], h2[1]])
    featy = jnp.stack([h1[2], h2[2], h1[3], h2[3]])
    px = _attn(featx, Wfc1x, bfc1x, Wfc2x, bfc2x, Wcx, bcx)
    py = _attn(featy, Wfc1y, bfc1y, Wfc2y, bfc2y, Wcy, bcy)
    return _mm(px, py.T)


# speedup vs baseline: 13.5231x; 1.7635x over previous
import functools

import jax
import jax.numpy as jnp
from jax import lax
from jax.experimental import pallas as pl
from jax.experimental.pallas import tpu as pltpu
from jax.experimental.pallas import tpu_sc as plsc

F = 128
N = 5000
NPAD = 5120
E = 160000
PADE = 163840
OC = 128
CH = 128
CPW = 40
EPW = PADE // 32
RPT = NPAD // 16


def _zero_vec(ref, nwords):
    def body(i, _):
        ref[pl.ds(i * 16, 16)] = jnp.zeros((16,), jnp.float32)
        return 0
    lax.fori_loop(0, nwords // 16, body, 0)


def _zero_rows(ref, nrows):
    def body(i, _):
        for j in range(F // 16):
            ref[i, pl.ds(j * 16, 16)] = jnp.zeros((16,), jnp.float32)
        return 0
    lax.fori_loop(0, nrows, body, 0)


def _vcopy(dst, src, off):
    for j in range(CH // 16):
        dst[pl.ds(j * 16, 16)] = src[pl.ds(off + j * 16, 16)]


@functools.cache
def _make_s1():
    mesh = plsc.VectorSubcoreMesh(core_axis_name="c", subcore_axis_name="s")
    return functools.partial(
        pl.kernel,
        mesh=mesh,
        out_type=[
            jax.ShapeDtypeStruct((4 * PADE,), jnp.float32),
            jax.ShapeDtypeStruct((8 * NPAD,), jnp.float32),
        ],
        scratch_types=[
            pltpu.VMEM((EPW,), jnp.int32),
            pltpu.VMEM((EPW,), jnp.int32),
            pltpu.VMEM((EPW,), jnp.int32),
            pltpu.VMEM((CH,), jnp.int32),
            pltpu.VMEM((CH,), jnp.int32),
            pltpu.VMEM((CH,), jnp.int32),
            pltpu.VMEM((CH,), jnp.int32),
            pltpu.VMEM((CH,), jnp.float32),
            pltpu.VMEM((CH,), jnp.float32),
            pltpu.VMEM((RPT,), jnp.float32),
            pltpu.VMEM_SHARED((NPAD,), jnp.float32),
            pltpu.VMEM_SHARED((NPAD,), jnp.float32),
            pltpu.VMEM_SHARED((NPAD,), jnp.float32),
            pltpu.VMEM_SHARED((NPAD,), jnp.float32),
            pltpu.SemaphoreType.DMA,
            pltpu.SemaphoreType.DMA,
            pltpu.SemaphoreType.DMA,
            pltpu.SemaphoreType.DMA,
            pltpu.SemaphoreType.DMA,
            pltpu.SemaphoreType.DMA,
        ],
    )(_s1_body)


def _s1_body(e0, m0, e1, m1, e2, m2, e3, m3,
             ew_out, deg_out,
             srcall, dstall, idxall, ib0, ib1, db0, db1, ewb0, ewb1, zv,
             dg0, dg1, dg2, dg3,
             gs0, gs1, ws0, ws1, ss0, ss1):
    ib = (ib0, ib1)
    db = (db0, db1)
    ewb = (ewb0, ewb1)
    gs = (gs0, gs1)
    ws = (ws0, ws1)
    ss = (ss0, ss1)
    deg_sh = (dg0, dg1, dg2, dg3)
    c = lax.axis_index("c")
    s = lax.axis_index("s")
    w = s * 2 + c

    _zero_vec(zv, RPT)
    for g in range(4):
        pltpu.sync_copy(zv, deg_sh[g].at[pl.ds(s * RPT, RPT)])
    plsc.subcore_barrier()

    for g, (ed, mat) in enumerate(((e0, m0), (e1, m1), (e2, m2), (e3, m3))):
        dg = deg_sh[g]
        goff = g * PADE + w * EPW

        pltpu.sync_copy(ed.at[pl.ds(w * EPW, EPW)], srcall)
        pltpu.sync_copy(ed.at[pl.ds(PADE + w * EPW, EPW)], dstall)

        def ib_body(k, _):
            sl = pl.ds(k * 16, 16)
            idxall[sl] = srcall[sl] * N + dstall[sl]
            return 0
        lax.fori_loop(0, EPW // 16, ib_body, 0)

        _vcopy(ib[0], idxall, 0)
        _vcopy(db[0], dstall, 0)
        pltpu.async_copy(mat.at[ib[0]], ewb[0], gs[0])

        def wait_pair(nb, goff=goff, mat=mat, dg=dg):
            pltpu.make_async_copy(
                ewb[nb], ew_out.at[pl.ds(goff, CH)], ws[nb]).wait()
            pltpu.make_async_copy(ewb[nb], dg.at[db[nb]], ss[nb]).wait()

        def prefetch(i, nb, mat=mat):
            off = (i + 1) * CH
            _vcopy(ib[nb], idxall, off)
            _vcopy(db[nb], dstall, off)
            pltpu.async_copy(mat.at[ib[nb]], ewb[nb], gs[nb])

        def it(i2, _, mat=mat, dg=dg, goff=goff):
            for b in (0, 1):
                i = i2 * 2 + b
                nb = 1 - b
                if b == 0:
                    @pl.when(i2 > 0)
                    def _():
                        wait_pair(nb)
                    prefetch(i, nb)
                else:
                    @pl.when(i2 < CPW // 2 - 1)
                    def _():
                        wait_pair(nb)
                        prefetch(i, nb)
                pltpu.make_async_copy(mat.at[ib[b]], ewb[b], gs[b]).wait()
                pltpu.async_copy(ewb[b],
                                 ew_out.at[pl.ds(goff + i * CH, CH)], ws[b])
                pltpu.async_copy(ewb[b], dg.at[db[b]], ss[b], add=True)
            return 0
        lax.fori_loop(0, CPW // 2, it, 0)
        wait_pair(0)
        wait_pair(1)

    plsc.subcore_barrier()
    for g in range(4):
        pltpu.sync_copy(deg_sh[g].at[pl.ds(s * RPT, RPT)], zv)
        pltpu.sync_copy(zv, deg_out.at[pl.ds((c * 4 + g) * NPAD + s * RPT,
                                             RPT)])


@functools.cache
def _make_s2():
    mesh = plsc.VectorSubcoreMesh(core_axis_name="c", subcore_axis_name="s")
    return functools.partial(
        pl.kernel,
        mesh=mesh,
        out_type=jax.ShapeDtypeStruct((2, 4, NPAD, F), jnp.float32),
        scratch_types=[
            pltpu.VMEM((EPW,), jnp.int32),
            pltpu.VMEM((EPW,), jnp.int32),
            pltpu.VMEM((EPW,), jnp.float32),
            pltpu.VMEM((CH,), jnp.int32),
            pltpu.VMEM((CH,), jnp.int32),
            pltpu.VMEM((CH,), jnp.int32),
            pltpu.VMEM((CH,), jnp.int32),
            pltpu.VMEM((CH, F), jnp.float32),
            pltpu.VMEM((CH, F), jnp.float32),
            pltpu.VMEM((RPT, F), jnp.float32),
            pltpu.VMEM_SHARED((NPAD, F), jnp.float32),
            pltpu.SemaphoreType.DMA,
            pltpu.SemaphoreType.DMA,
            pltpu.SemaphoreType.DMA,
            pltpu.SemaphoreType.DMA,
        ],
    )(_s2_body)


def _s2_body(tabs, e0, e1, e2, e3, ews,
             out,
             srcall, dstall, ewall, sb0, sb1, db0, db1, r0, r1, zrows,
             acc_sh, gs0, gs1, ss0, ss1):
    sb = (sb0, sb1)
    db = (db0, db1)
    rows = (r0, r1)
    gs = (gs0, gs1)
    ss = (ss0, ss1)
    c = lax.axis_index("c")
    s = lax.axis_index("s")
    w = s * 2 + c

    _zero_rows(zrows, RPT)
    pltpu.sync_copy(zrows, acc_sh.at[pl.ds(s * RPT, RPT)])
    plsc.subcore_barrier()

    for g, ed in enumerate((e0, e1, e2, e3)):
        tab = tabs.at[g]
        pltpu.sync_copy(ed.at[pl.ds(w * EPW, EPW)], srcall)
        pltpu.sync_copy(ed.at[pl.ds(PADE + w * EPW, EPW)], dstall)
        pltpu.sync_copy(ews.at[pl.ds(g * PADE + w * EPW, EPW)], ewall)

        _vcopy(sb[0], srcall, 0)
        _vcopy(db[0], dstall, 0)
        pltpu.async_copy(tab.at[sb[0]], rows[0], gs[0])

        def wait_sc(nb):
            pltpu.make_async_copy(rows[nb], acc_sh.at[db[nb]],
                                  ss[nb]).wait()

        def prefetch(i, nb, tab=tab):
            off = (i + 1) * CH
            _vcopy(sb[nb], srcall, off)
            _vcopy(db[nb], dstall, off)
            pltpu.async_copy(tab.at[sb[nb]], rows[nb], gs[nb])

        def it(i2, _, tab=tab):
            for b in (0, 1):
                i = i2 * 2 + b
                nb = 1 - b
                if b == 0:
                    @pl.when(i2 > 0)
                    def _():
                        wait_sc(nb)
                    prefetch(i, nb)
                else:
                    @pl.when(i2 < CPW // 2 - 1)
                    def _():
                        wait_sc(nb)
                        prefetch(i, nb)
                pltpu.make_async_copy(tab.at[sb[b]], rows[b], gs[b]).wait()

                rb = rows[b]

                def eb(k, _, rb=rb, i=i):
                    ew16 = ewall[pl.ds(i * CH + k * 16, 16)]
                    for l in range(16):
                        wv = jnp.full((16,), ew16[l], jnp.float32)
                        e = k * 16 + l
                        for j in range(F // 16):
                            sl = pl.ds(j * 16, 16)
                            rb[e, sl] = rb[e, sl] * wv
                    return 0
                lax.fori_loop(0, CH // 16, eb, 0)
                pltpu.async_copy(rows[b], acc_sh.at[db[b]], ss[b], add=True)
            return 0
        lax.fori_loop(0, CPW // 2, it, 0)
        wait_sc(0)
        wait_sc(1)

        plsc.subcore_barrier()
        pltpu.sync_copy(acc_sh.at[pl.ds(s * RPT, RPT)],
                        out.at[c, g, pl.ds(s * RPT, RPT)])
        if g < 3:
            pltpu.sync_copy(zrows, acc_sh.at[pl.ds(s * RPT, RPT)])
        plsc.subcore_barrier()


def _mm(a, b):
    M, K = a.shape
    Nn = b.shape[1]
    bm = 512
    bn = min(Nn, 512)

    def body(ar, br, orf):
        orf[...] = jnp.dot(ar[...], br[...],
                           preferred_element_type=jnp.float32)
    return pl.pallas_call(
        body,
        grid=(pl.cdiv(M, bm), pl.cdiv(Nn, bn)),
        in_specs=[
            pl.BlockSpec((bm, K), lambda i, j: (i, 0)),
            pl.BlockSpec((K, bn), lambda i, j: (0, j)),
        ],
        out_specs=pl.BlockSpec((bm, bn), lambda i, j: (i, j)),
        out_shape=jax.ShapeDtypeStruct((M, Nn), jnp.float32),
    )(a, b)


def _bmm(a, b):
    B, M, K = a.shape
    Nn = b.shape[2]
    bm = 512

    def body(ar, br, orf):
        orf[0] = jnp.dot(ar[0], br[0], preferred_element_type=jnp.float32)
    return pl.pallas_call(
        body,
        grid=(B, pl.cdiv(M, bm)),
        in_specs=[
            pl.BlockSpec((1, bm, K), lambda g, i: (g, i, 0)),
            pl.BlockSpec((1, K, Nn), lambda g, i: (g, 0, 0)),
        ],
        out_specs=pl.BlockSpec((1, bm, Nn), lambda g, i: (g, i, 0)),
        out_shape=jax.ShapeDtypeStruct((B, M, Nn), jnp.float32),
    )(a, b)


def _attn(feats4, Wfc1, bfc1, Wfc2, bfc2, Wc, bc):
    a = feats4.mean(axis=(1, 2))
    a = jax.nn.relu(a @ Wfc1.T + bfc1)
    a = jax.nn.sigmoid(a @ Wfc2.T + bfc2)
    A = (a[:, None, None] * Wc[..., 0].transpose(1, 2, 0)).reshape(4 * F, OC)
    Fcat = feats4.transpose(1, 0, 2).reshape(N, 4 * F)
    return _mm(Fcat, A) + bc[None, :]


def _pad_edges(e):
    k = PADE - E
    r = jnp.arange(k, dtype=jnp.int32)
    ps = (r * 97) % 4096
    pd = N + (r % (NPAD - N))
    return jnp.concatenate([e[0], ps, e[1], pd])


def kernel(mm_f_edges, mm_f_mat, mm_s_edges, mm_s_mat, dd_f_edges, dd_f_mat,
           dd_s_edges, dd_s_mat, x_m, x_d,
           Wx1f, bx1f, Wx2f, bx2f, Wx1s, bx1s, Wx2s, bx2s,
           Wy1f, by1f, Wy2f, by2f, Wy1s, by1s, Wy2s, by2s,
           Wfc1x, bfc1x, Wfc2x, bfc2x, Wfc1y, bfc1y, Wfc2y, bfc2y,
           Wcx, bcx, Wcy, bcy):
    edges = tuple(_pad_edges(e) for e in
                  (mm_f_edges, mm_s_edges, dd_f_edges, dd_s_edges))
    mats = (mm_f_mat.reshape(-1), mm_s_mat.reshape(-1),
            dd_f_mat.reshape(-1), dd_s_mat.reshape(-1))

    ew, degp = _make_s1()(edges[0], mats[0], edges[1], mats[1],
                          edges[2], mats[2], edges[3], mats[3])
    degp = degp.reshape(2, 4, NPAD)
    deg = degp[0] + degp[1] + 1.0
    dinv = jax.lax.rsqrt(deg)[:, :N]
    dcol = dinv[:, :, None]

    X0 = jnp.stack([x_m, x_m, x_d, x_d])
    W1 = jnp.stack([Wx1f, Wx1s, Wy1f, Wy1s]).transpose(0, 2, 1)
    b1 = jnp.stack([bx1f, bx1s, by1f, by1s])
    W2 = jnp.stack([Wx2f, Wx2s, Wy2f, Wy2s]).transpose(0, 2, 1)
    b2 = jnp.stack([bx2f, bx2s, by2f, by2s])

    def layer(xin, Wt, b):
        h = _bmm(xin, Wt)
        t = dcol * h
        aggp = _make_s2()(t, edges[0], edges[1], edges[2], edges[3], ew)
        agg = aggp[0, :, :N] + aggp[1, :, :N]
        return jax.nn.relu(dcol * (agg + t) + b[:, None, :])

    h1 = layer(X0, W1, b1)
    h2 = layer(h1, W2, b2)

    featx = jnp.stack([h1[0], h2[0], h1[1], h2[1]])
    featy = jnp.stack([h1[2], h2[2], h1[3], h2[3]])
    px = _attn(featx, Wfc1x, bfc1x, Wfc2x, bfc2x, Wcx, bcx)
    py = _attn(featy, Wfc1y, bfc1y, Wfc2y, bfc2y, Wcy, bcy)
    return _mm(px, py.T)
